# baseline (speedup 1.0000x reference)
import functools

import jax
import jax.numpy as jnp
from jax import lax
from jax.experimental import pallas as pl
from jax.experimental.pallas import tpu as pltpu
from jax.experimental.pallas import tpu_sc as plsc

BATCH = 16384
EMBED_DIM = 32
NUM_CORES = 2
NUM_SUBCORES = 16
LANES = 16
NUM_WORKERS = NUM_CORES * NUM_SUBCORES
B_PER_W = BATCH // NUM_WORKERS
GROUPS = B_PER_W // LANES


def _mf_kernel(user_hbm, mission_hbm, uemb_hbm, memb_hbm, ubias_hbm,
               mbias_hbm, bias_hbm, out_hbm,
               uidx_v, midx_v, urows_v, mrows_v, ub_v, mb_v, bidx_v, bias_v,
               out_v, sem_u, sem_m, sem_ub, sem_mb):
    wid = lax.axis_index("s") * NUM_CORES + lax.axis_index("c")
    base = wid * B_PER_W

    bidx_v[...] = jnp.zeros((LANES,), jnp.int32)
    pltpu.sync_copy(bias_hbm.at[bidx_v], bias_v)

    pltpu.sync_copy(user_hbm.at[pl.ds(base, B_PER_W)], uidx_v)
    pltpu.sync_copy(mission_hbm.at[pl.ds(base, B_PER_W)], midx_v)
    cp_u = pltpu.async_copy(uemb_hbm.at[uidx_v], urows_v, sem_u)
    cp_m = pltpu.async_copy(memb_hbm.at[midx_v], mrows_v, sem_m)
    cp_ub = pltpu.async_copy(ubias_hbm.at[uidx_v], ub_v, sem_ub)
    cp_mb = pltpu.async_copy(mbias_hbm.at[midx_v], mb_v, sem_mb)

    cp_u.wait()
    cp_m.wait()
    cp_ub.wait()
    cp_mb.wait()

    bias_val = bias_v[...]

    lane_iota = lax.iota(jnp.int32, LANES)

    def group_body(g, carry):
        off = g * LANES
        rows = off + lane_iota
        acc = ub_v[pl.ds(off, LANES)] + mb_v[pl.ds(off, LANES)] + bias_val
        for d in range(EMBED_DIM):
            col = jnp.full((LANES,), d, jnp.int32)
            uv = plsc.load_gather(urows_v, [rows, col])
            mv = plsc.load_gather(mrows_v, [rows, col])
            acc = acc + uv * mv
        out_v[pl.ds(off, LANES)] = acc
        return carry

    lax.fori_loop(0, GROUPS, group_body, 0)

    pltpu.sync_copy(out_v, out_hbm.at[pl.ds(base, B_PER_W)])


@jax.jit
def _run(user, mission, uemb, memb, ubias, mbias, bias):
    mesh = plsc.VectorSubcoreMesh(core_axis_name="c", subcore_axis_name="s")
    kfn = pl.kernel(
        _mf_kernel,
        out_type=jax.ShapeDtypeStruct((BATCH,), jnp.float32),
        mesh=mesh,
        compiler_params=pltpu.CompilerParams(needs_layout_passes=False,
                                             use_tc_tiling_on_sc=False),
        scratch_types=[
            pltpu.VMEM((B_PER_W,), jnp.int32),
            pltpu.VMEM((B_PER_W,), jnp.int32),
            pltpu.VMEM((B_PER_W, EMBED_DIM), jnp.float32),
            pltpu.VMEM((B_PER_W, EMBED_DIM), jnp.float32),
            pltpu.VMEM((B_PER_W,), jnp.float32),
            pltpu.VMEM((B_PER_W,), jnp.float32),
            pltpu.VMEM((LANES,), jnp.int32),
            pltpu.VMEM((LANES,), jnp.float32),
            pltpu.VMEM((B_PER_W,), jnp.float32),
            pltpu.SemaphoreType.DMA,
            pltpu.SemaphoreType.DMA,
            pltpu.SemaphoreType.DMA,
            pltpu.SemaphoreType.DMA,
        ],
    )
    return kfn(user, mission, uemb, memb, ubias, mbias, bias)


def kernel(user, mission, user_embedding, mission_embedding, user_bias,
           mission_bias, bias):
    user = user.astype(jnp.int32)
    mission = mission.astype(jnp.int32)
    return _run(user, mission, user_embedding, mission_embedding,
                user_bias.reshape(-1), mission_bias.reshape(-1),
                bias.reshape(-1))
